# baseline (speedup 1.0000x reference)
import functools

import jax
import jax.numpy as jnp
from jax import lax
from jax.experimental import pallas as pl
from jax.experimental.pallas import tpu as pltpu
from jax.experimental.pallas import tpu_sc as plsc

_INFO = plsc.get_sparse_core_info()
_NC = _INFO.num_cores
_NS = _INFO.num_subcores
_NW = _NC * _NS

_B = 16384
_F = 26
_D = 32
_C = 512
_L = 16

_mesh = lambda: plsc.VectorSubcoreMesh(core_axis_name="c",
                                       subcore_axis_name="s")


_V = 1000000
_W = 512
_BLK = 61


def _linearize(table_t):
    @functools.partial(
        pl.kernel,
        mesh=_mesh(),
        out_type=jax.ShapeDtypeStruct((_V * _D,), jnp.float32),
        scratch_types=[
            pltpu.VMEM((_D, _W), jnp.float32),
            pltpu.VMEM((_D, _W), jnp.float32),
            pltpu.VMEM((_W * _D,), jnp.float32),
            pltpu.VMEM((_W * _D,), jnp.float32),
            pltpu.SemaphoreType.DMA,
            pltpu.SemaphoreType.DMA,
            pltpu.SemaphoreType.DMA,
            pltpu.SemaphoreType.DMA,
        ],
        compiler_params=pltpu.CompilerParams(needs_layout_passes=False),
    )
    def k0(tab_hbm, out_hbm, in_a, in_b, rows_a, rows_b,
           si_a, si_b, so_a, so_b):
        wid = lax.axis_index("s") * _NC + lax.axis_index("c")
        c00 = wid * (_BLK * _W)
        in_v = (in_a, in_b)
        rows_v = (rows_a, rows_b)
        sem_i = (si_a, si_b)
        sem_o = (so_a, so_b)
        iota = lax.iota(jnp.int32, _L)
        ihi = iota + _L

        def ldd(c0, buf, width=_W):
            return pltpu.make_async_copy(tab_hbm.at[:, pl.ds(c0, width)],
                                         in_v[buf], sem_i[buf])

        def odd(c0, buf):
            return pltpu.make_async_copy(rows_v[buf],
                                         out_hbm.at[pl.ds(c0 * _D, _W * _D)],
                                         sem_o[buf])

        def transpose_block(buf, n_rows):
            inb = in_v[buf]
            rows = rows_v[buf]

            @plsc.parallel_loop(0, n_rows, step=1, unroll=8)
            def _(vl):
                cs = jnp.full((_L,), vl, jnp.int32)
                glo = plsc.load_gather(inb, [iota, cs])
                ghi = plsc.load_gather(inb, [ihi, cs])
                rows[pl.ds(vl * _D, _L)] = glo
                rows[pl.ds(vl * _D + _L, _L)] = ghi

        def process(c0, c0n, buf, skip_out_wait):
            ldd(c0n, 1 - buf).start()
            ldd(c0, buf).wait()

            @pl.when(jnp.logical_not(skip_out_wait))
            def _():
                odd(c0 - 2 * _W, buf).wait()

            transpose_block(buf, _W)
            odd(c0, buf).start()

        last = c00 + (_BLK - 1) * _W
        ldd(c00, 0).start()
        process(c00, c00 + _W, 0, True)

        def outer(g, _):
            c1 = c00 + (2 * g + 1) * _W
            process(c1, c1 + _W, 1, g == 0)
            c2 = c00 + (2 * g + 2) * _W
            process(c2, jnp.minimum(c2 + _W, last), 0, False)
            return 0

        lax.fori_loop(0, (_BLK - 1) // 2, outer, 0)
        ldd(last, 1).wait()
        odd(last - _W, 1).wait()
        odd(last, 0).wait()

        t0 = _BLK * _W * _NW

        @pl.when(wid == 0)
        def _():
            pltpu.sync_copy(tab_hbm.at[:, pl.ds(t0, _W)], in_a)
            transpose_block(0, _W)
            pltpu.sync_copy(rows_a, out_hbm.at[pl.ds(t0 * _D, _W * _D)])

    return k0(table_t)


def _gather(table, idx):
    n = idx.shape[0]
    n_per_w = n // _NW
    chunk = 1024
    n_chunks = n_per_w // chunk

    @functools.partial(
        pl.kernel,
        mesh=_mesh(),
        out_type=jax.ShapeDtypeStruct((n, _D), jnp.float32),
        scratch_types=[
            pltpu.VMEM((chunk,), jnp.int32),
            pltpu.VMEM((chunk,), jnp.int32),
            pltpu.VMEM((chunk, _D), jnp.float32),
            pltpu.VMEM((chunk, _D), jnp.float32),
            pltpu.SemaphoreType.DMA,
            pltpu.SemaphoreType.DMA,
            pltpu.SemaphoreType.DMA,
            pltpu.SemaphoreType.DMA,
        ],
        compiler_params=pltpu.CompilerParams(use_tc_tiling_on_sc=False),
    )
    def k1(table_hbm, idx_hbm, out_hbm, idx_a, idx_b, rows_a, rows_b,
           sem_ga, sem_gb, sem_oa, sem_ob):
        wid = lax.axis_index("s") * _NC + lax.axis_index("c")
        base = wid * n_per_w
        idx_v = (idx_a, idx_b)
        rows_v = (rows_a, rows_b)
        sem_g = (sem_ga, sem_gb)
        sem_o = (sem_oa, sem_ob)

        def stage(i):
            buf = i % 2
            pltpu.sync_copy(idx_hbm.at[pl.ds(base + i * chunk, chunk)],
                            idx_v[buf])
            return pltpu.async_copy(table_hbm.at[idx_v[buf]], rows_v[buf],
                                    sem_g[buf])

        gathers = {0: stage(0)}
        out_dmas = {}
        for i in range(n_chunks):
            buf = i % 2
            if i + 1 < n_chunks:
                gathers[i + 1] = stage(i + 1)
            gathers.pop(i).wait()
            if i - 2 in out_dmas:
                out_dmas.pop(i - 2).wait()
            out_dmas[i] = pltpu.async_copy(
                rows_v[buf], out_hbm.at[pl.ds(base + i * chunk, chunk)],
                sem_o[buf])
        for h in out_dmas.values():
            h.wait()

    return k1(table, idx)


def _reformat(rows_flat):
    @functools.partial(
        pl.kernel,
        mesh=_mesh(),
        out_type=jax.ShapeDtypeStruct((_F, _D // 8, _B // 128, 8, 128),
                                      jnp.float32),
        scratch_types=[
            pltpu.VMEM((_C * _D,), jnp.float32),
            pltpu.VMEM((_C * _D,), jnp.float32),
            pltpu.VMEM((_D // 8, _C // 128, 8, 128), jnp.float32),
            pltpu.VMEM((_D // 8, _C // 128, 8, 128), jnp.float32),
            pltpu.SemaphoreType.DMA,
            pltpu.SemaphoreType.DMA,
            pltpu.SemaphoreType.DMA,
            pltpu.SemaphoreType.DMA,
        ],
        compiler_params=pltpu.CompilerParams(needs_layout_passes=False),
    )
    def k2(rows_hbm, out_hbm, in_a, in_b, tp_a, tp_b,
           sem_ia, sem_ib, sem_oa, sem_ob):
        wid = lax.axis_index("s") * _NC + lax.axis_index("c")
        b0 = wid * _C
        bt0 = wid * (_C // 128)
        in_v = (in_a, in_b)
        tp_v = (tp_a, tp_b)
        sem_i = (sem_ia, sem_ib)
        sem_o = (sem_oa, sem_ob)

        iota = lax.iota(jnp.int32, _L)
        riota = iota * _D

        def load_desc(f, buf):
            off = (f * _B + b0) * _D
            return pltpu.make_async_copy(rows_hbm.at[pl.ds(off, _C * _D)],
                                         in_v[buf], sem_i[buf])

        def out_desc(f, buf):
            return pltpu.make_async_copy(
                tp_v[buf], out_hbm.at[f, :, pl.ds(bt0, _C // 128)],
                sem_o[buf])

        def process(f, buf, is_first):
            load_desc(jnp.minimum(f + 1, _F - 1), 1 - buf).start()
            load_desc(f, buf).wait()

            @pl.when(jnp.logical_not(is_first))
            def _():
                out_desc(f - 2, buf).wait()

            rows = in_v[buf]
            tp = tp_v[buf]

            @plsc.parallel_loop(0, _C // _L, step=1, unroll=2)
            def _(i):
                r0 = i * _L
                rvec = riota + r0 * _D
                bt = lax.shift_right_logical(r0, 7)
                bl = lax.bitwise_and(r0, 127)
                for d0 in range(0, _D, 8):
                    cols = [plsc.load_gather(rows, [rvec + (d0 + j)])
                            for j in range(8)]
                    for j in range(8):
                        d = d0 + j
                        tp[d // 8, bt, d % 8, pl.ds(bl, _L)] = cols[j]

            out_desc(f, buf).start()

        load_desc(0, 0).start()

        def outer(g, _):
            process(2 * g, 0, g == 0)
            process(2 * g + 1, 1, g == 0)
            return 0

        lax.fori_loop(0, _F // 2, outer, 0)
        load_desc(_F - 1, 0).wait()
        out_desc(_F - 2, 0).wait()
        out_desc(_F - 1, 1).wait()

    return k2(rows_flat)


def kernel(x, embedding_weight):
    b, f, _ = x.shape
    d = embedding_weight.shape[1]
    idx = jnp.transpose(jnp.squeeze(x, -1)).reshape(b * f)
    table_lin = _linearize(jnp.transpose(embedding_weight))
    t64 = (_BLK * _W * _NW + _W)
    table_lin = lax.dynamic_update_slice(
        table_lin, embedding_weight[t64:].reshape(-1), (t64 * d,))
    rows = _gather(table_lin.reshape(_V, d), idx)
    out5 = _reformat(rows.reshape(b * f * d))
    out = jnp.transpose(out5, (2, 4, 0, 1, 3)).reshape(b, f, d)
    return out[:, :, None, :]


# speedup vs baseline: 1.0767x; 1.0767x over previous
import functools

import jax
import jax.numpy as jnp
from jax import lax
from jax.experimental import pallas as pl
from jax.experimental.pallas import tpu as pltpu
from jax.experimental.pallas import tpu_sc as plsc

_INFO = plsc.get_sparse_core_info()
_NC = _INFO.num_cores
_NS = _INFO.num_subcores
_NW = _NC * _NS

_B = 16384
_F = 26
_D = 32
_C = 512
_L = 16

_mesh = lambda: plsc.VectorSubcoreMesh(core_axis_name="c",
                                       subcore_axis_name="s")


_V = 1000000
_W = 512
_BLK = 61


def _linearize(table_t):
    @functools.partial(
        pl.kernel,
        mesh=_mesh(),
        out_type=jax.ShapeDtypeStruct((_V * _D,), jnp.float32),
        scratch_types=[
            pltpu.VMEM((_D, _W), jnp.float32),
            pltpu.VMEM((_D, _W), jnp.float32),
            pltpu.VMEM((_W * _D,), jnp.float32),
            pltpu.VMEM((_W * _D,), jnp.float32),
            pltpu.SemaphoreType.DMA,
            pltpu.SemaphoreType.DMA,
            pltpu.SemaphoreType.DMA,
            pltpu.SemaphoreType.DMA,
        ],
        compiler_params=pltpu.CompilerParams(needs_layout_passes=False),
    )
    def k0(tab_hbm, out_hbm, in_a, in_b, rows_a, rows_b,
           si_a, si_b, so_a, so_b):
        wid = lax.axis_index("s") * _NC + lax.axis_index("c")
        c00 = wid * (_BLK * _W)
        in_v = (in_a, in_b)
        rows_v = (rows_a, rows_b)
        sem_i = (si_a, si_b)
        sem_o = (so_a, so_b)
        iota = lax.iota(jnp.int32, _L)
        ihi = iota + _L

        def ldd(c0, buf, width=_W):
            return pltpu.make_async_copy(tab_hbm.at[:, pl.ds(c0, width)],
                                         in_v[buf], sem_i[buf])

        def odd(c0, buf):
            return pltpu.make_async_copy(rows_v[buf],
                                         out_hbm.at[pl.ds(c0 * _D, _W * _D)],
                                         sem_o[buf])

        scat = iota * _D

        def transpose_block(buf, n_rows):
            inb = in_v[buf]
            rows = rows_v[buf]

            def tbody(i, _):
                c0 = i * _L
                base = scat + c0 * _D
                for d0 in range(0, _D, 8):
                    vals = [inb[d0 + j, pl.ds(c0, _L)] for j in range(8)]
                    for j in range(8):
                        plsc.store_scatter(rows, [base + (d0 + j)], vals[j])
                return 0

            lax.fori_loop(0, n_rows // _L, tbody, 0)

        def process(c0, c0n, buf, skip_out_wait):
            ldd(c0n, 1 - buf).start()
            ldd(c0, buf).wait()

            @pl.when(jnp.logical_not(skip_out_wait))
            def _():
                odd(c0 - 2 * _W, buf).wait()

            transpose_block(buf, _W)
            odd(c0, buf).start()

        last = c00 + (_BLK - 1) * _W
        ldd(c00, 0).start()
        process(c00, c00 + _W, 0, True)

        def outer(g, _):
            c1 = c00 + (2 * g + 1) * _W
            process(c1, c1 + _W, 1, g == 0)
            c2 = c00 + (2 * g + 2) * _W
            process(c2, jnp.minimum(c2 + _W, last), 0, False)
            return 0

        lax.fori_loop(0, (_BLK - 1) // 2, outer, 0)
        ldd(last, 1).wait()
        odd(last - _W, 1).wait()
        odd(last, 0).wait()

        t0 = _BLK * _W * _NW

        @pl.when(wid == 0)
        def _():
            pltpu.sync_copy(tab_hbm.at[:, pl.ds(t0, _W)], in_a)
            transpose_block(0, _W)
            pltpu.sync_copy(rows_a, out_hbm.at[pl.ds(t0 * _D, _W * _D)])

    return k0(table_t)


def _gather(table, idx):
    n = idx.shape[0]
    n_per_w = n // _NW
    chunk = 1024
    n_chunks = n_per_w // chunk

    @functools.partial(
        pl.kernel,
        mesh=_mesh(),
        out_type=jax.ShapeDtypeStruct((n, _D), jnp.float32),
        scratch_types=[
            pltpu.VMEM((chunk,), jnp.int32),
            pltpu.VMEM((chunk,), jnp.int32),
            pltpu.VMEM((chunk, _D), jnp.float32),
            pltpu.VMEM((chunk, _D), jnp.float32),
            pltpu.SemaphoreType.DMA,
            pltpu.SemaphoreType.DMA,
            pltpu.SemaphoreType.DMA,
            pltpu.SemaphoreType.DMA,
        ],
        compiler_params=pltpu.CompilerParams(use_tc_tiling_on_sc=False),
    )
    def k1(table_hbm, idx_hbm, out_hbm, idx_a, idx_b, rows_a, rows_b,
           sem_ga, sem_gb, sem_oa, sem_ob):
        wid = lax.axis_index("s") * _NC + lax.axis_index("c")
        base = wid * n_per_w
        idx_v = (idx_a, idx_b)
        rows_v = (rows_a, rows_b)
        sem_g = (sem_ga, sem_gb)
        sem_o = (sem_oa, sem_ob)

        def stage(i):
            buf = i % 2
            pltpu.sync_copy(idx_hbm.at[pl.ds(base + i * chunk, chunk)],
                            idx_v[buf])
            return pltpu.async_copy(table_hbm.at[idx_v[buf]], rows_v[buf],
                                    sem_g[buf])

        gathers = {0: stage(0)}
        out_dmas = {}
        for i in range(n_chunks):
            buf = i % 2
            if i + 1 < n_chunks:
                gathers[i + 1] = stage(i + 1)
            gathers.pop(i).wait()
            if i - 2 in out_dmas:
                out_dmas.pop(i - 2).wait()
            out_dmas[i] = pltpu.async_copy(
                rows_v[buf], out_hbm.at[pl.ds(base + i * chunk, chunk)],
                sem_o[buf])
        for h in out_dmas.values():
            h.wait()

    return k1(table, idx)


def _reformat(rows_flat):
    @functools.partial(
        pl.kernel,
        mesh=_mesh(),
        out_type=jax.ShapeDtypeStruct((_F, _D // 8, _B // 128, 8, 128),
                                      jnp.float32),
        scratch_types=[
            pltpu.VMEM((_C * _D,), jnp.float32),
            pltpu.VMEM((_C * _D,), jnp.float32),
            pltpu.VMEM((_D // 8, _C // 128, 8, 128), jnp.float32),
            pltpu.VMEM((_D // 8, _C // 128, 8, 128), jnp.float32),
            pltpu.SemaphoreType.DMA,
            pltpu.SemaphoreType.DMA,
            pltpu.SemaphoreType.DMA,
            pltpu.SemaphoreType.DMA,
        ],
        compiler_params=pltpu.CompilerParams(needs_layout_passes=False),
    )
    def k2(rows_hbm, out_hbm, in_a, in_b, tp_a, tp_b,
           sem_ia, sem_ib, sem_oa, sem_ob):
        wid = lax.axis_index("s") * _NC + lax.axis_index("c")
        b0 = wid * _C
        bt0 = wid * (_C // 128)
        in_v = (in_a, in_b)
        tp_v = (tp_a, tp_b)
        sem_i = (sem_ia, sem_ib)
        sem_o = (sem_oa, sem_ob)

        iota = lax.iota(jnp.int32, _L)
        riota = iota * _D

        def load_desc(f, buf):
            off = (f * _B + b0) * _D
            return pltpu.make_async_copy(rows_hbm.at[pl.ds(off, _C * _D)],
                                         in_v[buf], sem_i[buf])

        def out_desc(f, buf):
            return pltpu.make_async_copy(
                tp_v[buf], out_hbm.at[f, :, pl.ds(bt0, _C // 128)],
                sem_o[buf])

        def process(f, buf, is_first):
            load_desc(jnp.minimum(f + 1, _F - 1), 1 - buf).start()
            load_desc(f, buf).wait()

            @pl.when(jnp.logical_not(is_first))
            def _():
                out_desc(f - 2, buf).wait()

            rows = in_v[buf]
            tp = tp_v[buf]

            def body(i, _):
                r0 = i * _L
                rvec = riota + r0 * _D
                bt = lax.shift_right_logical(r0, 7)
                bl = lax.bitwise_and(r0, 127)
                for d0 in range(0, _D, 8):
                    cols = [plsc.load_gather(rows, [rvec + (d0 + j)])
                            for j in range(8)]
                    for j in range(8):
                        d = d0 + j
                        tp[d // 8, bt, d % 8, pl.ds(bl, _L)] = cols[j]
                return 0

            lax.fori_loop(0, _C // _L, body, 0)
            out_desc(f, buf).start()

        load_desc(0, 0).start()

        def outer(g, _):
            process(2 * g, 0, g == 0)
            process(2 * g + 1, 1, g == 0)
            return 0

        lax.fori_loop(0, _F // 2, outer, 0)
        load_desc(_F - 1, 0).wait()
        out_desc(_F - 2, 0).wait()
        out_desc(_F - 1, 1).wait()

    return k2(rows_flat)


def kernel(x, embedding_weight):
    b, f, _ = x.shape
    d = embedding_weight.shape[1]
    idx = jnp.transpose(jnp.squeeze(x, -1)).reshape(b * f)
    table_lin = _linearize(jnp.transpose(embedding_weight))
    t64 = (_BLK * _W * _NW + _W)
    table_lin = lax.dynamic_update_slice(
        table_lin, embedding_weight[t64:].reshape(-1), (t64 * d,))
    rows = _gather(table_lin.reshape(_V, d), idx)
    out5 = _reformat(rows.reshape(b * f * d))
    out = jnp.transpose(out5, (2, 4, 0, 1, 3)).reshape(b, f, d)
    return out[:, :, None, :]


# speedup vs baseline: 1.1201x; 1.0404x over previous
import functools

import jax
import jax.numpy as jnp
from jax import lax
from jax.experimental import pallas as pl
from jax.experimental.pallas import tpu as pltpu
from jax.experimental.pallas import tpu_sc as plsc

_INFO = plsc.get_sparse_core_info()
_NC = _INFO.num_cores
_NS = _INFO.num_subcores
_NW = _NC * _NS

_B = 16384
_F = 26
_D = 32
_C = 512
_L = 16

_mesh = lambda: plsc.VectorSubcoreMesh(core_axis_name="c",
                                       subcore_axis_name="s")


_V = 1000000
_W = 512
_BLK = 61


def _linearize(table_t):
    @functools.partial(
        pl.kernel,
        mesh=_mesh(),
        out_type=jax.ShapeDtypeStruct((_V * _D,), jnp.float32),
        scratch_types=[
            pltpu.VMEM((_D, _W), jnp.float32),
            pltpu.VMEM((_D, _W), jnp.float32),
            pltpu.VMEM((_W * _D,), jnp.float32),
            pltpu.VMEM((_W * _D,), jnp.float32),
            pltpu.SemaphoreType.DMA,
            pltpu.SemaphoreType.DMA,
            pltpu.SemaphoreType.DMA,
            pltpu.SemaphoreType.DMA,
        ],
        compiler_params=pltpu.CompilerParams(needs_layout_passes=False),
    )
    def k0(tab_hbm, out_hbm, in_a, in_b, rows_a, rows_b,
           si_a, si_b, so_a, so_b):
        wid = lax.axis_index("s") * _NC + lax.axis_index("c")
        c00 = wid * (_BLK * _W)
        in_v = (in_a, in_b)
        rows_v = (rows_a, rows_b)
        sem_i = (si_a, si_b)
        sem_o = (so_a, so_b)
        iota = lax.iota(jnp.int32, _L)
        ihi = iota + _L

        def ldd(c0, buf, width=_W):
            return pltpu.make_async_copy(tab_hbm.at[:, pl.ds(c0, width)],
                                         in_v[buf], sem_i[buf])

        def odd(c0, buf):
            return pltpu.make_async_copy(rows_v[buf],
                                         out_hbm.at[pl.ds(c0 * _D, _W * _D)],
                                         sem_o[buf])

        def transpose_block(buf, n_rows):
            inb = in_v[buf]
            rows = rows_v[buf]

            def tbody(i, _):
                cols = []
                for u in range(4):
                    vl = i * 4 + u
                    cs = jnp.full((_L,), vl, jnp.int32)
                    cols.append((vl,
                                 plsc.load_gather(inb, [iota, cs]),
                                 plsc.load_gather(inb, [ihi, cs])))
                for vl, glo, ghi in cols:
                    rows[pl.ds(vl * _D, _L)] = glo
                    rows[pl.ds(vl * _D + _L, _L)] = ghi
                return 0

            lax.fori_loop(0, n_rows // 4, tbody, 0)

        def process(c0, c0n, buf, skip_out_wait):
            ldd(c0n, 1 - buf).start()
            ldd(c0, buf).wait()

            @pl.when(jnp.logical_not(skip_out_wait))
            def _():
                odd(c0 - 2 * _W, buf).wait()

            transpose_block(buf, _W)
            odd(c0, buf).start()

        last = c00 + (_BLK - 1) * _W
        ldd(c00, 0).start()
        process(c00, c00 + _W, 0, True)

        def outer(g, _):
            c1 = c00 + (2 * g + 1) * _W
            process(c1, c1 + _W, 1, g == 0)
            c2 = c00 + (2 * g + 2) * _W
            process(c2, jnp.minimum(c2 + _W, last), 0, False)
            return 0

        lax.fori_loop(0, (_BLK - 1) // 2, outer, 0)
        ldd(last, 1).wait()
        odd(last - _W, 1).wait()
        odd(last, 0).wait()

        t0 = _BLK * _W * _NW

        @pl.when(wid == 0)
        def _():
            pltpu.sync_copy(tab_hbm.at[:, pl.ds(t0, _W)], in_a)
            transpose_block(0, _W)
            pltpu.sync_copy(rows_a, out_hbm.at[pl.ds(t0 * _D, _W * _D)])

    return k0(table_t)


def _gather(table, idx):
    n = idx.shape[0]
    n_per_w = n // _NW
    chunk = 1024
    n_chunks = n_per_w // chunk

    @functools.partial(
        pl.kernel,
        mesh=_mesh(),
        out_type=jax.ShapeDtypeStruct((n, _D), jnp.float32),
        scratch_types=[
            pltpu.VMEM((chunk,), jnp.int32),
            pltpu.VMEM((chunk,), jnp.int32),
            pltpu.VMEM((chunk, _D), jnp.float32),
            pltpu.VMEM((chunk, _D), jnp.float32),
            pltpu.SemaphoreType.DMA,
            pltpu.SemaphoreType.DMA,
            pltpu.SemaphoreType.DMA,
            pltpu.SemaphoreType.DMA,
        ],
        compiler_params=pltpu.CompilerParams(use_tc_tiling_on_sc=False),
    )
    def k1(table_hbm, idx_hbm, out_hbm, idx_a, idx_b, rows_a, rows_b,
           sem_ga, sem_gb, sem_oa, sem_ob):
        wid = lax.axis_index("s") * _NC + lax.axis_index("c")
        base = wid * n_per_w
        idx_v = (idx_a, idx_b)
        rows_v = (rows_a, rows_b)
        sem_g = (sem_ga, sem_gb)
        sem_o = (sem_oa, sem_ob)

        def stage(i):
            buf = i % 2
            pltpu.sync_copy(idx_hbm.at[pl.ds(base + i * chunk, chunk)],
                            idx_v[buf])
            return pltpu.async_copy(table_hbm.at[idx_v[buf]], rows_v[buf],
                                    sem_g[buf])

        gathers = {0: stage(0)}
        out_dmas = {}
        for i in range(n_chunks):
            buf = i % 2
            if i + 1 < n_chunks:
                gathers[i + 1] = stage(i + 1)
            gathers.pop(i).wait()
            if i - 2 in out_dmas:
                out_dmas.pop(i - 2).wait()
            out_dmas[i] = pltpu.async_copy(
                rows_v[buf], out_hbm.at[pl.ds(base + i * chunk, chunk)],
                sem_o[buf])
        for h in out_dmas.values():
            h.wait()

    return k1(table, idx)


def _reformat(rows_flat):
    @functools.partial(
        pl.kernel,
        mesh=_mesh(),
        out_type=jax.ShapeDtypeStruct((_F, _D // 8, _B // 128, 8, 128),
                                      jnp.float32),
        scratch_types=[
            pltpu.VMEM((_C * _D,), jnp.float32),
            pltpu.VMEM((_C * _D,), jnp.float32),
            pltpu.VMEM((_D // 8, _C // 128, 8, 128), jnp.float32),
            pltpu.VMEM((_D // 8, _C // 128, 8, 128), jnp.float32),
            pltpu.SemaphoreType.DMA,
            pltpu.SemaphoreType.DMA,
            pltpu.SemaphoreType.DMA,
            pltpu.SemaphoreType.DMA,
        ],
        compiler_params=pltpu.CompilerParams(needs_layout_passes=False),
    )
    def k2(rows_hbm, out_hbm, in_a, in_b, tp_a, tp_b,
           sem_ia, sem_ib, sem_oa, sem_ob):
        wid = lax.axis_index("s") * _NC + lax.axis_index("c")
        b0 = wid * _C
        bt0 = wid * (_C // 128)
        in_v = (in_a, in_b)
        tp_v = (tp_a, tp_b)
        sem_i = (sem_ia, sem_ib)
        sem_o = (sem_oa, sem_ob)

        iota = lax.iota(jnp.int32, _L)
        riota = iota * _D

        def load_desc(f, buf):
            off = (f * _B + b0) * _D
            return pltpu.make_async_copy(rows_hbm.at[pl.ds(off, _C * _D)],
                                         in_v[buf], sem_i[buf])

        def out_desc(f, buf):
            return pltpu.make_async_copy(
                tp_v[buf], out_hbm.at[f, :, pl.ds(bt0, _C // 128)],
                sem_o[buf])

        def process(f, buf, is_first):
            load_desc(jnp.minimum(f + 1, _F - 1), 1 - buf).start()
            load_desc(f, buf).wait()

            @pl.when(jnp.logical_not(is_first))
            def _():
                out_desc(f - 2, buf).wait()

            rows = in_v[buf]
            tp = tp_v[buf]

            def body(i, _):
                r0 = i * _L
                rvec = riota + r0 * _D
                bt = lax.shift_right_logical(r0, 7)
                bl = lax.bitwise_and(r0, 127)
                for d0 in range(0, _D, 8):
                    cols = [plsc.load_gather(rows, [rvec + (d0 + j)])
                            for j in range(8)]
                    for j in range(8):
                        d = d0 + j
                        tp[d // 8, bt, d % 8, pl.ds(bl, _L)] = cols[j]
                return 0

            lax.fori_loop(0, _C // _L, body, 0)
            out_desc(f, buf).start()

        load_desc(0, 0).start()

        def outer(g, _):
            process(2 * g, 0, g == 0)
            process(2 * g + 1, 1, g == 0)
            return 0

        lax.fori_loop(0, _F // 2, outer, 0)
        load_desc(_F - 1, 0).wait()
        out_desc(_F - 2, 0).wait()
        out_desc(_F - 1, 1).wait()

    return k2(rows_flat)


def kernel(x, embedding_weight):
    b, f, _ = x.shape
    d = embedding_weight.shape[1]
    idx = jnp.transpose(jnp.squeeze(x, -1)).reshape(b * f)
    table_lin = _linearize(jnp.transpose(embedding_weight))
    t64 = (_BLK * _W * _NW + _W)
    table_lin = lax.dynamic_update_slice(
        table_lin, embedding_weight[t64:].reshape(-1), (t64 * d,))
    rows = _gather(table_lin.reshape(_V, d), idx)
    out5 = _reformat(rows.reshape(b * f * d))
    out = jnp.transpose(out5, (2, 4, 0, 1, 3)).reshape(b, f, d)
    return out[:, :, None, :]


# speedup vs baseline: 1.5455x; 1.3797x over previous
import functools

import jax
import jax.numpy as jnp
from jax import lax
from jax.experimental import pallas as pl
from jax.experimental.pallas import tpu as pltpu
from jax.experimental.pallas import tpu_sc as plsc

_INFO = plsc.get_sparse_core_info()
_NC = _INFO.num_cores
_NS = _INFO.num_subcores
_NW = _NC * _NS

_B = 16384
_F = 26
_D = 32
_C = 512
_L = 16

_mesh = lambda: plsc.VectorSubcoreMesh(core_axis_name="c",
                                       subcore_axis_name="s")


_V = 1000000
_W = 512
_BLK = 61


def _linearize(table_t):
    @functools.partial(
        pl.kernel,
        mesh=_mesh(),
        out_type=jax.ShapeDtypeStruct((_V * _D,), jnp.float32),
        scratch_types=[
            pltpu.VMEM((_D, _W), jnp.float32),
            pltpu.VMEM((_D, _W), jnp.float32),
            pltpu.VMEM((_W * _D,), jnp.float32),
            pltpu.VMEM((_W * _D,), jnp.float32),
            pltpu.SemaphoreType.DMA,
            pltpu.SemaphoreType.DMA,
            pltpu.SemaphoreType.DMA,
            pltpu.SemaphoreType.DMA,
        ],
        compiler_params=pltpu.CompilerParams(needs_layout_passes=False),
    )
    def k0(tab_hbm, out_hbm, in_a, in_b, rows_a, rows_b,
           si_a, si_b, so_a, so_b):
        wid = lax.axis_index("s") * _NC + lax.axis_index("c")
        c00 = wid * (_BLK * _W)
        in_v = (in_a, in_b)
        rows_v = (rows_a, rows_b)
        sem_i = (si_a, si_b)
        sem_o = (so_a, so_b)
        iota = lax.iota(jnp.int32, _L)
        ihi = iota + _L

        def ldd(c0, buf, width=_W):
            return pltpu.make_async_copy(tab_hbm.at[:, pl.ds(c0, width)],
                                         in_v[buf], sem_i[buf])

        def odd(c0, buf):
            return pltpu.make_async_copy(rows_v[buf],
                                         out_hbm.at[pl.ds(c0 * _D, _W * _D)],
                                         sem_o[buf])

        dwvecs = [lax.bitwise_and(iota + d0, _D - 1) for d0 in range(_D)]

        def transpose_block(buf, n_rows):
            inb = in_v[buf]
            rows = rows_v[buf]

            def tbody(i, _):
                cvec = iota + i * _L
                cbase = cvec * _D
                for d0 in range(_D):
                    x = plsc.load_gather(inb, [dwvecs[d0], cvec])
                    plsc.store_scatter(rows, [cbase + dwvecs[d0]], x)
                return 0

            lax.fori_loop(0, n_rows // _L, tbody, 0)

        def process(c0, c0n, buf, skip_out_wait):
            ldd(c0n, 1 - buf).start()
            ldd(c0, buf).wait()

            @pl.when(jnp.logical_not(skip_out_wait))
            def _():
                odd(c0 - 2 * _W, buf).wait()

            transpose_block(buf, _W)
            odd(c0, buf).start()

        last = c00 + (_BLK - 1) * _W
        ldd(c00, 0).start()
        process(c00, c00 + _W, 0, True)

        def outer(g, _):
            c1 = c00 + (2 * g + 1) * _W
            process(c1, c1 + _W, 1, g == 0)
            c2 = c00 + (2 * g + 2) * _W
            process(c2, jnp.minimum(c2 + _W, last), 0, False)
            return 0

        lax.fori_loop(0, (_BLK - 1) // 2, outer, 0)
        ldd(last, 1).wait()
        odd(last - _W, 1).wait()
        odd(last, 0).wait()

        t0 = _BLK * _W * _NW

        @pl.when(wid == 0)
        def _():
            pltpu.sync_copy(tab_hbm.at[:, pl.ds(t0, _W)], in_a)
            transpose_block(0, _W)
            pltpu.sync_copy(rows_a, out_hbm.at[pl.ds(t0 * _D, _W * _D)])

    return k0(table_t)


def _gather(table, idx):
    n = idx.shape[0]
    n_per_w = n // _NW
    chunk = 1024
    n_chunks = n_per_w // chunk

    @functools.partial(
        pl.kernel,
        mesh=_mesh(),
        out_type=jax.ShapeDtypeStruct((n, _D), jnp.float32),
        scratch_types=[
            pltpu.VMEM((chunk,), jnp.int32),
            pltpu.VMEM((chunk,), jnp.int32),
            pltpu.VMEM((chunk, _D), jnp.float32),
            pltpu.VMEM((chunk, _D), jnp.float32),
            pltpu.SemaphoreType.DMA,
            pltpu.SemaphoreType.DMA,
            pltpu.SemaphoreType.DMA,
            pltpu.SemaphoreType.DMA,
        ],
        compiler_params=pltpu.CompilerParams(use_tc_tiling_on_sc=False),
    )
    def k1(table_hbm, idx_hbm, out_hbm, idx_a, idx_b, rows_a, rows_b,
           sem_ga, sem_gb, sem_oa, sem_ob):
        wid = lax.axis_index("s") * _NC + lax.axis_index("c")
        base = wid * n_per_w
        idx_v = (idx_a, idx_b)
        rows_v = (rows_a, rows_b)
        sem_g = (sem_ga, sem_gb)
        sem_o = (sem_oa, sem_ob)

        def stage(i):
            buf = i % 2
            pltpu.sync_copy(idx_hbm.at[pl.ds(base + i * chunk, chunk)],
                            idx_v[buf])
            return pltpu.async_copy(table_hbm.at[idx_v[buf]], rows_v[buf],
                                    sem_g[buf])

        gathers = {0: stage(0)}
        out_dmas = {}
        for i in range(n_chunks):
            buf = i % 2
            if i + 1 < n_chunks:
                gathers[i + 1] = stage(i + 1)
            gathers.pop(i).wait()
            if i - 2 in out_dmas:
                out_dmas.pop(i - 2).wait()
            out_dmas[i] = pltpu.async_copy(
                rows_v[buf], out_hbm.at[pl.ds(base + i * chunk, chunk)],
                sem_o[buf])
        for h in out_dmas.values():
            h.wait()

    return k1(table, idx)


def _reformat(rows_flat):
    @functools.partial(
        pl.kernel,
        mesh=_mesh(),
        out_type=jax.ShapeDtypeStruct((_F, _D // 8, _B // 128, 8, 128),
                                      jnp.float32),
        scratch_types=[
            pltpu.VMEM((_C * _D,), jnp.float32),
            pltpu.VMEM((_C * _D,), jnp.float32),
            pltpu.VMEM((_D // 8, _C // 128, 8, 128), jnp.float32),
            pltpu.VMEM((_D // 8, _C // 128, 8, 128), jnp.float32),
            pltpu.SemaphoreType.DMA,
            pltpu.SemaphoreType.DMA,
            pltpu.SemaphoreType.DMA,
            pltpu.SemaphoreType.DMA,
        ],
        compiler_params=pltpu.CompilerParams(needs_layout_passes=False),
    )
    def k2(rows_hbm, out_hbm, in_a, in_b, tp_a, tp_b,
           sem_ia, sem_ib, sem_oa, sem_ob):
        wid = lax.axis_index("s") * _NC + lax.axis_index("c")
        b0 = wid * _C
        bt0 = wid * (_C // 128)
        in_v = (in_a, in_b)
        tp_v = (tp_a, tp_b)
        sem_i = (sem_ia, sem_ib)
        sem_o = (sem_oa, sem_ob)

        iota = lax.iota(jnp.int32, _L)
        riota = iota * _D

        def load_desc(f, buf):
            off = (f * _B + b0) * _D
            return pltpu.make_async_copy(rows_hbm.at[pl.ds(off, _C * _D)],
                                         in_v[buf], sem_i[buf])

        def out_desc(f, buf):
            return pltpu.make_async_copy(
                tp_v[buf], out_hbm.at[f, :, pl.ds(bt0, _C // 128)],
                sem_o[buf])

        def process(f, buf, is_first):
            load_desc(jnp.minimum(f + 1, _F - 1), 1 - buf).start()
            load_desc(f, buf).wait()

            @pl.when(jnp.logical_not(is_first))
            def _():
                out_desc(f - 2, buf).wait()

            rows = in_v[buf]
            tp = tp_v[buf]

            def body(i, _):
                r0 = i * _L
                rvec = riota + r0 * _D
                bt = lax.shift_right_logical(r0, 7)
                bl = lax.bitwise_and(r0, 127)
                for d0 in range(0, _D, 8):
                    cols = [plsc.load_gather(rows, [rvec + (d0 + j)])
                            for j in range(8)]
                    for j in range(8):
                        d = d0 + j
                        tp[d // 8, bt, d % 8, pl.ds(bl, _L)] = cols[j]
                return 0

            lax.fori_loop(0, _C // _L, body, 0)
            out_desc(f, buf).start()

        load_desc(0, 0).start()

        def outer(g, _):
            process(2 * g, 0, g == 0)
            process(2 * g + 1, 1, g == 0)
            return 0

        lax.fori_loop(0, _F // 2, outer, 0)
        load_desc(_F - 1, 0).wait()
        out_desc(_F - 2, 0).wait()
        out_desc(_F - 1, 1).wait()

    return k2(rows_flat)


def kernel(x, embedding_weight):
    b, f, _ = x.shape
    d = embedding_weight.shape[1]
    idx = jnp.transpose(jnp.squeeze(x, -1)).reshape(b * f)
    table_lin = _linearize(jnp.transpose(embedding_weight))
    t64 = (_BLK * _W * _NW + _W)
    table_lin = lax.dynamic_update_slice(
        table_lin, embedding_weight[t64:].reshape(-1), (t64 * d,))
    rows = _gather(table_lin.reshape(_V, d), idx)
    out5 = _reformat(rows.reshape(b * f * d))
    out = jnp.transpose(out5, (2, 4, 0, 1, 3)).reshape(b, f, d)
    return out[:, :, None, :]


# speedup vs baseline: 1.8535x; 1.1993x over previous
import functools

import jax
import jax.numpy as jnp
from jax import lax
from jax.experimental import pallas as pl
from jax.experimental.pallas import tpu as pltpu
from jax.experimental.pallas import tpu_sc as plsc

_INFO = plsc.get_sparse_core_info()
_NC = _INFO.num_cores
_NS = _INFO.num_subcores
_NW = _NC * _NS

_B = 16384
_F = 26
_D = 32
_C = 512
_L = 16

_mesh = lambda: plsc.VectorSubcoreMesh(core_axis_name="c",
                                       subcore_axis_name="s")


_V = 1000000
_W = 512
_BLK = 61


def _linearize(table_t):
    @functools.partial(
        pl.kernel,
        mesh=_mesh(),
        out_type=jax.ShapeDtypeStruct((_V * _D,), jnp.float32),
        scratch_types=[
            pltpu.VMEM((_D, _W), jnp.float32),
            pltpu.VMEM((_D, _W), jnp.float32),
            pltpu.VMEM((_W * _D,), jnp.float32),
            pltpu.VMEM((_W * _D,), jnp.float32),
            pltpu.SemaphoreType.DMA,
            pltpu.SemaphoreType.DMA,
            pltpu.SemaphoreType.DMA,
            pltpu.SemaphoreType.DMA,
        ],
        compiler_params=pltpu.CompilerParams(needs_layout_passes=False),
    )
    def k0(tab_hbm, out_hbm, in_a, in_b, rows_a, rows_b,
           si_a, si_b, so_a, so_b):
        wid = lax.axis_index("s") * _NC + lax.axis_index("c")
        c00 = wid * (_BLK * _W)
        in_v = (in_a, in_b)
        rows_v = (rows_a, rows_b)
        sem_i = (si_a, si_b)
        sem_o = (so_a, so_b)
        iota = lax.iota(jnp.int32, _L)
        ihi = iota + _L

        def ldd(c0, buf, width=_W):
            return pltpu.make_async_copy(tab_hbm.at[:, pl.ds(c0, width)],
                                         in_v[buf], sem_i[buf])

        def odd(c0, buf):
            return pltpu.make_async_copy(rows_v[buf],
                                         out_hbm.at[pl.ds(c0 * _D, _W * _D)],
                                         sem_o[buf])

        dwvecs = [lax.bitwise_and(iota + d0, _D - 1) for d0 in range(_D)]

        def transpose_block(buf, n_rows):
            inb = in_v[buf]
            rows = rows_v[buf]

            def tbody(i, _):
                cvec = iota + i * _L
                cbase = cvec * _D
                for d0 in range(_D):
                    x = plsc.load_gather(inb, [dwvecs[d0], cvec])
                    plsc.store_scatter(rows, [cbase + dwvecs[d0]], x)
                return 0

            lax.fori_loop(0, n_rows // _L, tbody, 0)

        def process(c0, c0n, buf, skip_out_wait):
            ldd(c0n, 1 - buf).start()
            ldd(c0, buf).wait()

            @pl.when(jnp.logical_not(skip_out_wait))
            def _():
                odd(c0 - 2 * _W, buf).wait()

            transpose_block(buf, _W)
            odd(c0, buf).start()

        last = c00 + (_BLK - 1) * _W
        ldd(c00, 0).start()
        process(c00, c00 + _W, 0, True)

        def outer(g, _):
            c1 = c00 + (2 * g + 1) * _W
            process(c1, c1 + _W, 1, g == 0)
            c2 = c00 + (2 * g + 2) * _W
            process(c2, jnp.minimum(c2 + _W, last), 0, False)
            return 0

        lax.fori_loop(0, (_BLK - 1) // 2, outer, 0)
        ldd(last, 1).wait()
        odd(last - _W, 1).wait()
        odd(last, 0).wait()

        t0 = _BLK * _W * _NW

        @pl.when(wid == 0)
        def _():
            pltpu.sync_copy(tab_hbm.at[:, pl.ds(t0, _W)], in_a)
            transpose_block(0, _W)
            pltpu.sync_copy(rows_a, out_hbm.at[pl.ds(t0 * _D, _W * _D)])

    return k0(table_t)


def _gather(table, idx):
    n = idx.shape[0]
    n_per_w = n // _NW
    chunk = 1024
    n_chunks = n_per_w // chunk

    @functools.partial(
        pl.kernel,
        mesh=_mesh(),
        out_type=jax.ShapeDtypeStruct((n, _D), jnp.float32),
        scratch_types=[
            pltpu.VMEM((chunk,), jnp.int32),
            pltpu.VMEM((chunk,), jnp.int32),
            pltpu.VMEM((chunk, _D), jnp.float32),
            pltpu.VMEM((chunk, _D), jnp.float32),
            pltpu.SemaphoreType.DMA,
            pltpu.SemaphoreType.DMA,
            pltpu.SemaphoreType.DMA,
            pltpu.SemaphoreType.DMA,
        ],
        compiler_params=pltpu.CompilerParams(use_tc_tiling_on_sc=False),
    )
    def k1(table_hbm, idx_hbm, out_hbm, idx_a, idx_b, rows_a, rows_b,
           sem_ga, sem_gb, sem_oa, sem_ob):
        wid = lax.axis_index("s") * _NC + lax.axis_index("c")
        base = wid * n_per_w
        idx_v = (idx_a, idx_b)
        rows_v = (rows_a, rows_b)
        sem_g = (sem_ga, sem_gb)
        sem_o = (sem_oa, sem_ob)

        def stage(i):
            buf = i % 2
            pltpu.sync_copy(idx_hbm.at[pl.ds(base + i * chunk, chunk)],
                            idx_v[buf])
            return pltpu.async_copy(table_hbm.at[idx_v[buf]], rows_v[buf],
                                    sem_g[buf])

        gathers = {0: stage(0)}
        out_dmas = {}
        for i in range(n_chunks):
            buf = i % 2
            if i + 1 < n_chunks:
                gathers[i + 1] = stage(i + 1)
            gathers.pop(i).wait()
            if i - 2 in out_dmas:
                out_dmas.pop(i - 2).wait()
            out_dmas[i] = pltpu.async_copy(
                rows_v[buf], out_hbm.at[pl.ds(base + i * chunk, chunk)],
                sem_o[buf])
        for h in out_dmas.values():
            h.wait()

    return k1(table, idx)


def _reformat(rows_flat):
    @functools.partial(
        pl.kernel,
        mesh=_mesh(),
        out_type=jax.ShapeDtypeStruct((_F, _D // 8, _B // 128, 8, 128),
                                      jnp.float32),
        scratch_types=[
            pltpu.VMEM((_C * _D,), jnp.float32),
            pltpu.VMEM((_C * _D,), jnp.float32),
            pltpu.VMEM((_D // 8, _C // 128, 8, 128), jnp.float32),
            pltpu.VMEM((_D // 8, _C // 128, 8, 128), jnp.float32),
            pltpu.SemaphoreType.DMA,
            pltpu.SemaphoreType.DMA,
            pltpu.SemaphoreType.DMA,
            pltpu.SemaphoreType.DMA,
        ],
        compiler_params=pltpu.CompilerParams(needs_layout_passes=False),
    )
    def k2(rows_hbm, out_hbm, in_a, in_b, tp_a, tp_b,
           sem_ia, sem_ib, sem_oa, sem_ob):
        wid = lax.axis_index("s") * _NC + lax.axis_index("c")
        b0 = wid * _C
        bt0 = wid * (_C // 128)
        in_v = (in_a, in_b)
        tp_v = (tp_a, tp_b)
        sem_i = (sem_ia, sem_ib)
        sem_o = (sem_oa, sem_ob)

        iota = lax.iota(jnp.int32, _L)
        dwvecs = [lax.bitwise_and(iota + d0, _D - 1) for d0 in range(_D)]
        kkvecs = [lax.shift_right_logical(v, 3) for v in dwvecs]
        svecs = [lax.bitwise_and(v, 7) for v in dwvecs]

        def load_desc(f, buf):
            off = (f * _B + b0) * _D
            return pltpu.make_async_copy(rows_hbm.at[pl.ds(off, _C * _D)],
                                         in_v[buf], sem_i[buf])

        def out_desc(f, buf):
            return pltpu.make_async_copy(
                tp_v[buf], out_hbm.at[f, :, pl.ds(bt0, _C // 128)],
                sem_o[buf])

        def process(f, buf, is_first):
            load_desc(jnp.minimum(f + 1, _F - 1), 1 - buf).start()
            load_desc(f, buf).wait()

            @pl.when(jnp.logical_not(is_first))
            def _():
                out_desc(f - 2, buf).wait()

            rows = in_v[buf]
            tp = tp_v[buf]

            def body(i, _):
                rvec = iota + i * _L
                rbase = rvec * _D
                btv = lax.shift_right_logical(rvec, 7)
                blv = lax.bitwise_and(rvec, 127)
                for d0 in range(_D):
                    x = plsc.load_gather(rows, [rbase + dwvecs[d0]])
                    plsc.store_scatter(tp, [kkvecs[d0], btv, svecs[d0], blv],
                                       x)
                return 0

            lax.fori_loop(0, _C // _L, body, 0)
            out_desc(f, buf).start()

        load_desc(0, 0).start()

        def outer(g, _):
            process(2 * g, 0, g == 0)
            process(2 * g + 1, 1, g == 0)
            return 0

        lax.fori_loop(0, _F // 2, outer, 0)
        load_desc(_F - 1, 0).wait()
        out_desc(_F - 2, 0).wait()
        out_desc(_F - 1, 1).wait()

    return k2(rows_flat)


def kernel(x, embedding_weight):
    b, f, _ = x.shape
    d = embedding_weight.shape[1]
    idx = jnp.transpose(jnp.squeeze(x, -1)).reshape(b * f)
    table_lin = _linearize(jnp.transpose(embedding_weight))
    t64 = (_BLK * _W * _NW + _W)
    table_lin = lax.dynamic_update_slice(
        table_lin, embedding_weight[t64:].reshape(-1), (t64 * d,))
    rows = _gather(table_lin.reshape(_V, d), idx)
    out5 = _reformat(rows.reshape(b * f * d))
    out = jnp.transpose(out5, (2, 4, 0, 1, 3)).reshape(b, f, d)
    return out[:, :, None, :]
